# baseline (speedup 1.0000x reference)
import jax
import jax.numpy as jnp
from jax import lax
from jax.experimental import pallas as pl
from jax.experimental.pallas import tpu as pltpu
from jax.experimental.pallas import tpu_sc as plsc

_C = 1000
_P = 16
_D = 512
_ROWS = _C * _P
_NC = 2
_NS = 16
_NW = _NC * _NS

_TC_BLK = 125
_TC_NCHUNK = _C // _TC_BLK


def _sc_idx_body(idx_hbm, idx_v):
    wid = lax.axis_index("s") * _NC + lax.axis_index("c")
    for i in range(32):
        idx_v[pl.ds(_P * i, _P)] = jnp.full((_P,), 32 * wid + i, jnp.int32)

    @pl.when(wid < _NW - 1)
    def _():
        pltpu.sync_copy(idx_v, idx_hbm.at[pl.ds(512 * wid, 512)])

    @pl.when(wid == _NW - 1)
    def _():
        pltpu.sync_copy(idx_v.at[pl.ds(0, 128)],
                        idx_hbm.at[pl.ds(512 * (_NW - 1), 128)])


def _tc_copy_body(in_any, out_any, buf, rsem, wsem):
    def rd(k):
        return pltpu.make_async_copy(in_any.at[pl.ds(k * _TC_BLK, _TC_BLK)],
                                     buf.at[k], rsem.at[k])

    def wr(k):
        return pltpu.make_async_copy(buf.at[k],
                                     out_any.at[pl.ds(k * _TC_BLK, _TC_BLK)],
                                     wsem.at[k])

    for k in range(_TC_NCHUNK):
        rd(k).start()
    for k in range(_TC_NCHUNK):
        rd(k).wait()
        wr(k).start()
    for k in range(_TC_NCHUNK):
        wr(k).wait()


def kernel(vision_protos):
    class_idx = pl.kernel(
        _sc_idx_body,
        out_type=jax.ShapeDtypeStruct((_ROWS,), jnp.int32),
        mesh=plsc.VectorSubcoreMesh(core_axis_name="c", subcore_axis_name="s"),
        scratch_types=[pltpu.VMEM((512,), jnp.int32)],
    )()

    stacked = pl.pallas_call(
        _tc_copy_body,
        in_specs=[pl.BlockSpec(memory_space=pl.ANY)],
        out_specs=pl.BlockSpec(memory_space=pl.ANY),
        out_shape=jax.ShapeDtypeStruct((_C, _P, _D), jnp.float32),
        scratch_shapes=[
            pltpu.VMEM((_TC_NCHUNK, _TC_BLK, _P, _D), jnp.float32),
            pltpu.SemaphoreType.DMA((_TC_NCHUNK,)),
            pltpu.SemaphoreType.DMA((_TC_NCHUNK,)),
        ],
    )(vision_protos)

    return (stacked.reshape(_ROWS, _D), class_idx)


# speedup vs baseline: 1.6833x; 1.6833x over previous
import jax
import jax.numpy as jnp
from jax import lax
from jax.experimental import pallas as pl
from jax.experimental.pallas import tpu as pltpu
from jax.experimental.pallas import tpu_sc as plsc

_C = 1000
_P = 16
_D = 512
_ROWS = _C * _P
_NC = 2
_NS = 16
_NW = _NC * _NS

_TC_BLK = 125
_TC_NCHUNK = _C // _TC_BLK


def _sc_idx_body(idx_hbm, idx_v):
    wid = lax.axis_index("s") * _NC + lax.axis_index("c")
    for i in range(32):
        idx_v[pl.ds(_P * i, _P)] = jnp.full((_P,), 32 * wid + i, jnp.int32)

    @pl.when(wid < _NW - 1)
    def _():
        pltpu.sync_copy(idx_v, idx_hbm.at[pl.ds(512 * wid, 512)])

    @pl.when(wid == _NW - 1)
    def _():
        pltpu.sync_copy(idx_v.at[pl.ds(0, 128)],
                        idx_hbm.at[pl.ds(512 * (_NW - 1), 128)])


def _tc_copy_body(in_any, out_any, idx_ref, buf, rsem, wsem):
    def rd(k):
        return pltpu.make_async_copy(in_any.at[pl.ds(k * _TC_BLK, _TC_BLK)],
                                     buf.at[k], rsem.at[k])

    def wr(k):
        return pltpu.make_async_copy(buf.at[k],
                                     out_any.at[pl.ds(k * _TC_BLK, _TC_BLK)],
                                     wsem.at[k])

    for k in range(_TC_NCHUNK):
        rd(k).start()
    i = lax.broadcasted_iota(jnp.int32, (125, 128), 0)
    j = lax.broadcasted_iota(jnp.int32, (125, 128), 1)
    idx_ref[...] = (i * 128 + j) >> 4
    for k in range(_TC_NCHUNK):
        rd(k).wait()
        wr(k).start()
    for k in range(_TC_NCHUNK):
        wr(k).wait()


def kernel(vision_protos):
    stacked, idx2d = pl.pallas_call(
        _tc_copy_body,
        in_specs=[pl.BlockSpec(memory_space=pl.ANY)],
        out_specs=[pl.BlockSpec(memory_space=pl.ANY),
                   pl.BlockSpec((125, 128), lambda: (0, 0))],
        out_shape=[jax.ShapeDtypeStruct((_C, _P, _D), jnp.float32),
                   jax.ShapeDtypeStruct((125, 128), jnp.int32)],
        scratch_shapes=[
            pltpu.VMEM((_TC_NCHUNK, _TC_BLK, _P, _D), jnp.float32),
            pltpu.SemaphoreType.DMA((_TC_NCHUNK,)),
            pltpu.SemaphoreType.DMA((_TC_NCHUNK,)),
        ],
    )(vision_protos)

    return (stacked.reshape(_ROWS, _D), idx2d.reshape(_ROWS))
